# baseline (speedup 1.0000x reference)
import functools

import jax
import jax.numpy as jnp
from jax import lax
from jax.experimental import pallas as pl
from jax.experimental.pallas import tpu as pltpu
from jax.experimental.pallas import tpu_sc as plsc

_VOCAB = 1000000
_DIM = 32
_HIDDEN = 128
_CLA = 10
_B = 4096
_L = 200
_EPS = 1e-5

_CV = 16384
_CQ = _CV // 4
_GRID = -(-_VOCAB // _CV)
_VPAD = _GRID * _CV

_NC = 2
_NS = 16
_NW = _NC * _NS
_BPW = _B // _NW
_C0 = 128
_C1 = _L - _C0
_NBUF = 4
_NIDX = _BPW * _L

_SHG = _CV.bit_length() - 1
_SHA = _CQ.bit_length() - 1
_MSKR = _CQ - 1


def _tr_body(in_ref, sel_ref, out_ref):
    x = in_ref[...]
    acc = None
    for a in range(4):
        p = jax.lax.dot_general(
            x[:, a * _CQ:(a + 1) * _CQ], sel_ref[a * _DIM:(a + 1) * _DIM, :],
            (((0,), (0,)), ((), ())), precision=jax.lax.Precision.HIGHEST,
            preferred_element_type=jnp.float32)
        acc = p if acc is None else acc + p
    out_ref[...] = acc


def _transpose_table(embT):
    sel = jnp.eye(128, dtype=jnp.float32)
    return pl.pallas_call(
        _tr_body,
        grid=(_GRID,),
        in_specs=[
            pl.BlockSpec((_DIM, _CV), lambda k: (0, k)),
            pl.BlockSpec((4 * _DIM, 128), lambda k: (0, 0)),
        ],
        out_specs=pl.BlockSpec((_CQ, 128), lambda k: (k, 0)),
        out_shape=jax.ShapeDtypeStruct((_GRID * _CQ, 128), jnp.float32),
    )(embT, sel)


def _sc_pool_body(x_hbm, tbl_hbm, out_hbm, idx_v, rows_v, acc_v, sems):
    wid = lax.axis_index("s") * _NC + lax.axis_index("c")
    base = wid * _BPW
    pltpu.sync_copy(x_hbm.at[pl.ds(base * _L, _NIDX)], idx_v)

    def tr_idx(k, carry):
        v = idx_v[pl.ds(k * 16, 16)]
        j = ((v >> _SHG) << _SHG) + ((v & _MSKR) << 2) + ((v >> _SHA) & 3)
        idx_v[pl.ds(k * 16, 16)] = j
        return carry

    lax.fori_loop(0, _NIDX // 16, tr_idx, 0, unroll=8)

    def fire(row, b):
        i0 = row * _L
        pltpu.async_copy(
            tbl_hbm.at[idx_v.at[pl.ds(i0, _C0)]],
            rows_v.at[b].at[pl.ds(0, _C0)], sems.at[b])
        pltpu.async_copy(
            tbl_hbm.at[idx_v.at[pl.ds(i0 + _C0, _C1)]],
            rows_v.at[b].at[pl.ds(_C0, _C1)], sems.at[b])

    def wait(b):
        pltpu.make_async_copy(
            tbl_hbm.at[pl.ds(0, _L)], rows_v.at[b], sems.at[b]).wait()

    def accum(r, b):
        def acc_body(j, accs):
            a0, a1 = accs
            return (a0 + rows_v[b, j, pl.ds(0, 16)],
                    a1 + rows_v[b, j, pl.ds(16, 16)])

        z = jnp.zeros((16,), jnp.float32)
        a0, a1 = lax.fori_loop(0, _L, acc_body, (z, z), unroll=8)
        acc_v[r, pl.ds(0, 16)] = a0
        acc_v[r, pl.ds(16, 16)] = a1

    for b in range(_NBUF):
        fire(b, b)

    def group_body(g, carry):
        for b in range(_NBUF):
            r = g * _NBUF + b
            wait(b)
            accum(r, b)

            @pl.when(r + _NBUF < _BPW)
            def _():
                fire(r + _NBUF, b)
        return carry

    lax.fori_loop(0, _BPW // _NBUF, group_body, 0)
    pltpu.sync_copy(acc_v, out_hbm.at[pl.ds(base, _BPW)])


_sc_pool = functools.partial(
    pl.kernel,
    mesh=plsc.VectorSubcoreMesh(core_axis_name="c", subcore_axis_name="s"),
    out_type=jax.ShapeDtypeStruct((_B, _DIM), jnp.float32),
    compiler_params=pltpu.CompilerParams(use_tc_tiling_on_sc=False),
    scratch_types=[
        pltpu.VMEM((_NIDX,), jnp.int32),
        pltpu.VMEM((_NBUF, _L, _DIM), jnp.float32),
        pltpu.VMEM((_BPW, _DIM), jnp.float32),
        pltpu.SemaphoreType.DMA((_NBUF,)),
    ],
)(_sc_pool_body)


def _mlp_body(m_ref, w1_ref, b1_ref, g_ref, bt_ref, w2_ref, b2_ref, o_ref):
    m = m_ref[...] * (1.0 / _L)
    h = jax.lax.dot_general(
        m, w1_ref[...], (((1,), (0,)), ((), ())),
        preferred_element_type=jnp.float32)
    h = h + b1_ref[...]
    mu = jnp.mean(h, axis=0, keepdims=True)
    d = h - mu
    var = jnp.mean(d * d, axis=0, keepdims=True)
    hn = d * lax.rsqrt(var + _EPS) * g_ref[...] + bt_ref[...]
    hr = jnp.maximum(hn, 0.0)
    o_ref[...] = jax.lax.dot_general(
        hr, w2_ref[...], (((1,), (0,)), ((), ())),
        preferred_element_type=jnp.float32) + b2_ref[...]


def kernel(x, emb, W1, b1, gamma, beta, W2, b2):
    xf = jnp.reshape(x.astype(jnp.int32), (_B * _L,))
    table = _transpose_table(jnp.transpose(emb))
    tblv = jnp.reshape(table, (_VPAD, _DIM))
    msum = _sc_pool(xf, tblv)
    logit = pl.pallas_call(
        _mlp_body,
        out_shape=jax.ShapeDtypeStruct((_B, _CLA), jnp.float32),
    )(msum, W1, b1.reshape(1, _HIDDEN), gamma.reshape(1, _HIDDEN),
      beta.reshape(1, _HIDDEN), W2, b2.reshape(1, _CLA))
    return logit


# speedup vs baseline: 1.8012x; 1.8012x over previous
import functools

import jax
import jax.numpy as jnp
from jax import lax
from jax.experimental import pallas as pl
from jax.experimental.pallas import tpu as pltpu
from jax.experimental.pallas import tpu_sc as plsc

_VOCAB = 1000000
_DIM = 32
_HIDDEN = 128
_CLA = 10
_B = 4096
_L = 200
_EPS = 1e-5

_CV = 16384
_CQ = _CV // 4
_GRID = -(-_VOCAB // _CV)
_VPAD = _GRID * _CV

_NC = 2
_NS = 16
_NW = _NC * _NS
_BPW = _B // _NW
_C0 = 128
_C1 = _L - _C0
_NBUF = 4
_NIDX = _BPW * _L

_SHG = _CV.bit_length() - 1
_SHA = _CQ.bit_length() - 1
_MSKR = _CQ - 1


def _tr_body(in_ref, out_ref):
    x = in_ref[...]
    parts = [x[:, a * _CQ:(a + 1) * _CQ].T for a in range(4)]
    out_ref[...] = jnp.concatenate(parts, axis=1)


def _transpose_table(embT):
    return pl.pallas_call(
        _tr_body,
        grid=(_GRID,),
        in_specs=[pl.BlockSpec((_DIM, _CV), lambda k: (0, k))],
        out_specs=pl.BlockSpec((_CQ, 128), lambda k: (k, 0)),
        out_shape=jax.ShapeDtypeStruct((_GRID * _CQ, 128), jnp.float32),
    )(embT)


def _sc_pool_body(x_hbm, tbl_hbm, out_hbm, idx_v, rows_v, acc_v, sems):
    wid = lax.axis_index("s") * _NC + lax.axis_index("c")
    base = wid * _BPW
    pltpu.sync_copy(x_hbm.at[pl.ds(base * _L, _NIDX)], idx_v)

    def tr_idx(k, carry):
        v = idx_v[pl.ds(k * 16, 16)]
        j = ((v >> _SHG) << _SHG) + ((v & _MSKR) << 2) + ((v >> _SHA) & 3)
        idx_v[pl.ds(k * 16, 16)] = j
        return carry

    lax.fori_loop(0, _NIDX // 16, tr_idx, 0, unroll=8)

    def fire(row, b):
        i0 = row * _L
        pltpu.async_copy(
            tbl_hbm.at[idx_v.at[pl.ds(i0, _C0)]],
            rows_v.at[b].at[pl.ds(0, _C0)], sems.at[b])
        pltpu.async_copy(
            tbl_hbm.at[idx_v.at[pl.ds(i0 + _C0, _C1)]],
            rows_v.at[b].at[pl.ds(_C0, _C1)], sems.at[b])

    def wait(b):
        pltpu.make_async_copy(
            tbl_hbm.at[pl.ds(0, _L)], rows_v.at[b], sems.at[b]).wait()

    def accum(r, b):
        def acc_body(j, accs):
            a0, a1 = accs
            return (a0 + rows_v[b, j, pl.ds(0, 16)],
                    a1 + rows_v[b, j, pl.ds(16, 16)])

        z = jnp.zeros((16,), jnp.float32)
        a0, a1 = lax.fori_loop(0, _L, acc_body, (z, z), unroll=8)
        acc_v[r, pl.ds(0, 16)] = a0
        acc_v[r, pl.ds(16, 16)] = a1

    for b in range(_NBUF):
        fire(b, b)

    def group_body(g, carry):
        for b in range(_NBUF):
            r = g * _NBUF + b
            wait(b)
            accum(r, b)

            @pl.when(r + _NBUF < _BPW)
            def _():
                fire(r + _NBUF, b)
        return carry

    lax.fori_loop(0, _BPW // _NBUF, group_body, 0)
    pltpu.sync_copy(acc_v, out_hbm.at[pl.ds(base, _BPW)])


_sc_pool = functools.partial(
    pl.kernel,
    mesh=plsc.VectorSubcoreMesh(core_axis_name="c", subcore_axis_name="s"),
    out_type=jax.ShapeDtypeStruct((_B, _DIM), jnp.float32),
    compiler_params=pltpu.CompilerParams(use_tc_tiling_on_sc=False),
    scratch_types=[
        pltpu.VMEM((_NIDX,), jnp.int32),
        pltpu.VMEM((_NBUF, _L, _DIM), jnp.float32),
        pltpu.VMEM((_BPW, _DIM), jnp.float32),
        pltpu.SemaphoreType.DMA((_NBUF,)),
    ],
)(_sc_pool_body)


def _mlp_body(m_ref, w1_ref, b1_ref, g_ref, bt_ref, w2_ref, b2_ref, o_ref):
    m = m_ref[...] * (1.0 / _L)
    h = jax.lax.dot_general(
        m, w1_ref[...], (((1,), (0,)), ((), ())),
        preferred_element_type=jnp.float32)
    h = h + b1_ref[...]
    mu = jnp.mean(h, axis=0, keepdims=True)
    d = h - mu
    var = jnp.mean(d * d, axis=0, keepdims=True)
    hn = d * lax.rsqrt(var + _EPS) * g_ref[...] + bt_ref[...]
    hr = jnp.maximum(hn, 0.0)
    o_ref[...] = jax.lax.dot_general(
        hr, w2_ref[...], (((1,), (0,)), ((), ())),
        preferred_element_type=jnp.float32) + b2_ref[...]


def kernel(x, emb, W1, b1, gamma, beta, W2, b2):
    xf = jnp.reshape(x.astype(jnp.int32), (_B * _L,))
    table = _transpose_table(jnp.transpose(emb))
    tblv = jnp.reshape(table, (_VPAD, _DIM))
    msum = _sc_pool(xf, tblv)
    logit = pl.pallas_call(
        _mlp_body,
        out_shape=jax.ShapeDtypeStruct((_B, _CLA), jnp.float32),
    )(msum, W1, b1.reshape(1, _HIDDEN), gamma.reshape(1, _HIDDEN),
      beta.reshape(1, _HIDDEN), W2, b2.reshape(1, _CLA))
    return logit


# speedup vs baseline: 3.2181x; 1.7866x over previous
import functools

import jax
import jax.numpy as jnp
from jax import lax
from jax.experimental import pallas as pl
from jax.experimental.pallas import tpu as pltpu
from jax.experimental.pallas import tpu_sc as plsc

_VOCAB = 1000000
_DIM = 32
_HIDDEN = 128
_CLA = 10
_B = 4096
_L = 200
_EPS = 1e-5

_CV = 16384
_CQ = _CV // 4
_GRID = -(-_VOCAB // _CV)
_VPAD = _GRID * _CV

_NC = 2
_NS = 16
_NW = _NC * _NS
_BPW = _B // _NW
_C0 = 128
_C1 = _L - _C0
_NBUF = 4
_NIDX = _BPW * _L


def _tr_body(in_ref, out_ref, stg_ref):
    for u in range(_CV // 512):
        s = stg_ref.at[u % 2]
        for q in range(4):
            c0 = 512 * u + 128 * q
            s[32 * q:32 * q + 32, :] = in_ref[:, c0:c0 + 128]
        out_ref[128 * u:128 * u + 128, :] = s[...].T


def _transpose_table(embT):
    return pl.pallas_call(
        _tr_body,
        grid=(_GRID,),
        in_specs=[pl.BlockSpec((_DIM, _CV), lambda k: (0, k))],
        out_specs=pl.BlockSpec((_CQ, 128), lambda k: (k, 0)),
        out_shape=jax.ShapeDtypeStruct((_GRID * _CQ, 128), jnp.float32),
        scratch_shapes=[pltpu.VMEM((2, 128, 128), jnp.float32)],
    )(embT)


def _sc_pool_body(x_hbm, tbl_hbm, out_hbm, idx_v, rows_v, acc_v, sems):
    wid = lax.axis_index("s") * _NC + lax.axis_index("c")
    base = wid * _BPW
    pltpu.sync_copy(x_hbm.at[pl.ds(base * _L, _NIDX)], idx_v)

    def tr_idx(k, carry):
        v = idx_v[pl.ds(k * 16, 16)]
        j = ((v >> 9) << 9) + ((v & 127) << 2) + ((v >> 7) & 3)
        idx_v[pl.ds(k * 16, 16)] = j
        return carry

    lax.fori_loop(0, _NIDX // 16, tr_idx, 0, unroll=8)

    def fire(row, b):
        i0 = row * _L
        pltpu.async_copy(
            tbl_hbm.at[idx_v.at[pl.ds(i0, _C0)]],
            rows_v.at[b].at[pl.ds(0, _C0)], sems.at[b])
        pltpu.async_copy(
            tbl_hbm.at[idx_v.at[pl.ds(i0 + _C0, _C1)]],
            rows_v.at[b].at[pl.ds(_C0, _C1)], sems.at[b])

    def wait(b):
        pltpu.make_async_copy(
            tbl_hbm.at[pl.ds(0, _L)], rows_v.at[b], sems.at[b]).wait()

    def accum(r, b):
        def acc_body(j, accs):
            a0, a1 = accs
            return (a0 + rows_v[b, j, pl.ds(0, 16)],
                    a1 + rows_v[b, j, pl.ds(16, 16)])

        z = jnp.zeros((16,), jnp.float32)
        a0, a1 = lax.fori_loop(0, _L, acc_body, (z, z), unroll=8)
        acc_v[r, pl.ds(0, 16)] = a0
        acc_v[r, pl.ds(16, 16)] = a1

    for b in range(_NBUF):
        fire(b, b)

    def group_body(g, carry):
        for b in range(_NBUF):
            r = g * _NBUF + b
            wait(b)
            accum(r, b)

            @pl.when(r + _NBUF < _BPW)
            def _():
                fire(r + _NBUF, b)
        return carry

    lax.fori_loop(0, _BPW // _NBUF, group_body, 0)
    pltpu.sync_copy(acc_v, out_hbm.at[pl.ds(base, _BPW)])


_sc_pool = functools.partial(
    pl.kernel,
    mesh=plsc.VectorSubcoreMesh(core_axis_name="c", subcore_axis_name="s"),
    out_type=jax.ShapeDtypeStruct((_B, _DIM), jnp.float32),
    compiler_params=pltpu.CompilerParams(use_tc_tiling_on_sc=False),
    scratch_types=[
        pltpu.VMEM((_NIDX,), jnp.int32),
        pltpu.VMEM((_NBUF, _L, _DIM), jnp.float32),
        pltpu.VMEM((_BPW, _DIM), jnp.float32),
        pltpu.SemaphoreType.DMA((_NBUF,)),
    ],
)(_sc_pool_body)


def _mlp_body(m_ref, w1_ref, b1_ref, g_ref, bt_ref, w2_ref, b2_ref, o_ref):
    m = m_ref[...] * (1.0 / _L)
    h = jax.lax.dot_general(
        m, w1_ref[...], (((1,), (0,)), ((), ())),
        preferred_element_type=jnp.float32)
    h = h + b1_ref[...]
    mu = jnp.mean(h, axis=0, keepdims=True)
    d = h - mu
    var = jnp.mean(d * d, axis=0, keepdims=True)
    hn = d * lax.rsqrt(var + _EPS) * g_ref[...] + bt_ref[...]
    hr = jnp.maximum(hn, 0.0)
    o_ref[...] = jax.lax.dot_general(
        hr, w2_ref[...], (((1,), (0,)), ((), ())),
        preferred_element_type=jnp.float32) + b2_ref[...]


def kernel(x, emb, W1, b1, gamma, beta, W2, b2):
    xf = jnp.reshape(x.astype(jnp.int32), (_B * _L,))
    table = _transpose_table(jnp.transpose(emb))
    tblv = jnp.reshape(table, (_VPAD, _DIM))
    msum = _sc_pool(xf, tblv)
    logit = pl.pallas_call(
        _mlp_body,
        out_shape=jax.ShapeDtypeStruct((_B, _CLA), jnp.float32),
    )(msum, W1, b1.reshape(1, _HIDDEN), gamma.reshape(1, _HIDDEN),
      beta.reshape(1, _HIDDEN), W2, b2.reshape(1, _CLA))
    return logit


# speedup vs baseline: 3.4926x; 1.0853x over previous
import functools

import jax
import jax.numpy as jnp
from jax import lax
from jax.experimental import pallas as pl
from jax.experimental.pallas import tpu as pltpu
from jax.experimental.pallas import tpu_sc as plsc

_VOCAB = 1000000
_DIM = 32
_HIDDEN = 128
_CLA = 10
_B = 4096
_L = 200
_EPS = 1e-5

_CV = 32768
_CQ = _CV // 4
_GRID = -(-_VOCAB // _CV)
_VPAD = _GRID * _CV

_NC = 2
_NS = 16
_NW = _NC * _NS
_BPW = _B // _NW
_C0 = 128
_C1 = _L - _C0
_NBUF = 4
_NIDX = _BPW * _L


def _tr_body(in_ref, out_ref, stg_ref):
    for u in range(_CV // 512):
        s = stg_ref.at[u % 2]
        for q in range(4):
            c0 = 512 * u + 128 * q
            s[32 * q:32 * q + 32, :] = in_ref[:, c0:c0 + 128]
        out_ref[128 * u:128 * u + 128, :] = s[...].T


def _transpose_table(embT):
    return pl.pallas_call(
        _tr_body,
        grid=(_GRID,),
        in_specs=[pl.BlockSpec((_DIM, _CV), lambda k: (0, k))],
        out_specs=pl.BlockSpec((_CQ, 128), lambda k: (k, 0)),
        out_shape=jax.ShapeDtypeStruct((_GRID * _CQ, 128), jnp.float32),
        scratch_shapes=[pltpu.VMEM((2, 128, 128), jnp.float32)],
    )(embT)


def _sc_pool_body(x_hbm, tbl_hbm, out_hbm, idx_v, rows_v, acc_v, sems):
    wid = lax.axis_index("s") * _NC + lax.axis_index("c")
    base = wid * _BPW
    pltpu.sync_copy(x_hbm.at[pl.ds(base * _L, _NIDX)], idx_v)

    def tr_idx(k, carry):
        v = idx_v[pl.ds(k * 16, 16)]
        j = ((v >> 9) << 9) + ((v & 127) << 2) + ((v >> 7) & 3)
        idx_v[pl.ds(k * 16, 16)] = j
        return carry

    lax.fori_loop(0, _NIDX // 16, tr_idx, 0, unroll=8)

    def fire(row, b):
        i0 = row * _L
        pltpu.async_copy(
            tbl_hbm.at[idx_v.at[pl.ds(i0, _C0)]],
            rows_v.at[b].at[pl.ds(0, _C0)], sems.at[b])
        pltpu.async_copy(
            tbl_hbm.at[idx_v.at[pl.ds(i0 + _C0, _C1)]],
            rows_v.at[b].at[pl.ds(_C0, _C1)], sems.at[b])

    def wait(b):
        pltpu.make_async_copy(
            tbl_hbm.at[pl.ds(0, _L)], rows_v.at[b], sems.at[b]).wait()

    def accum(r, b):
        def acc_body(j, accs):
            a0, a1 = accs
            return (a0 + rows_v[b, j, pl.ds(0, 16)],
                    a1 + rows_v[b, j, pl.ds(16, 16)])

        z = jnp.zeros((16,), jnp.float32)
        a0, a1 = lax.fori_loop(0, _L, acc_body, (z, z), unroll=8)
        acc_v[r, pl.ds(0, 16)] = a0
        acc_v[r, pl.ds(16, 16)] = a1

    for b in range(_NBUF):
        fire(b, b)

    def group_body(g, carry):
        for b in range(_NBUF):
            r = g * _NBUF + b
            wait(b)
            accum(r, b)

            @pl.when(r + _NBUF < _BPW)
            def _():
                fire(r + _NBUF, b)
        return carry

    lax.fori_loop(0, _BPW // _NBUF, group_body, 0)
    pltpu.sync_copy(acc_v, out_hbm.at[pl.ds(base, _BPW)])


_sc_pool = functools.partial(
    pl.kernel,
    mesh=plsc.VectorSubcoreMesh(core_axis_name="c", subcore_axis_name="s"),
    out_type=jax.ShapeDtypeStruct((_B, _DIM), jnp.float32),
    compiler_params=pltpu.CompilerParams(use_tc_tiling_on_sc=False),
    scratch_types=[
        pltpu.VMEM((_NIDX,), jnp.int32),
        pltpu.VMEM((_NBUF, _L, _DIM), jnp.float32),
        pltpu.VMEM((_BPW, _DIM), jnp.float32),
        pltpu.SemaphoreType.DMA((_NBUF,)),
    ],
)(_sc_pool_body)


def _mlp_body(m_ref, w1_ref, b1_ref, g_ref, bt_ref, w2_ref, b2_ref, o_ref):
    m = m_ref[...] * (1.0 / _L)
    h = jax.lax.dot_general(
        m, w1_ref[...], (((1,), (0,)), ((), ())),
        preferred_element_type=jnp.float32)
    h = h + b1_ref[...]
    mu = jnp.mean(h, axis=0, keepdims=True)
    d = h - mu
    var = jnp.mean(d * d, axis=0, keepdims=True)
    hn = d * lax.rsqrt(var + _EPS) * g_ref[...] + bt_ref[...]
    hr = jnp.maximum(hn, 0.0)
    o_ref[...] = jax.lax.dot_general(
        hr, w2_ref[...], (((1,), (0,)), ((), ())),
        preferred_element_type=jnp.float32) + b2_ref[...]


def kernel(x, emb, W1, b1, gamma, beta, W2, b2):
    xf = jnp.reshape(x.astype(jnp.int32), (_B * _L,))
    table = _transpose_table(jnp.transpose(emb))
    tblv = jnp.reshape(table, (_VPAD, _DIM))
    msum = _sc_pool(xf, tblv)
    logit = pl.pallas_call(
        _mlp_body,
        out_shape=jax.ShapeDtypeStruct((_B, _CLA), jnp.float32),
    )(msum, W1, b1.reshape(1, _HIDDEN), gamma.reshape(1, _HIDDEN),
      beta.reshape(1, _HIDDEN), W2, b2.reshape(1, _CLA))
    return logit


# speedup vs baseline: 3.5245x; 1.0091x over previous
import functools

import jax
import jax.numpy as jnp
from jax import lax
from jax.experimental import pallas as pl
from jax.experimental.pallas import tpu as pltpu
from jax.experimental.pallas import tpu_sc as plsc

_VOCAB = 1000000
_DIM = 32
_HIDDEN = 128
_CLA = 10
_B = 4096
_L = 200
_EPS = 1e-5

_CV = 65536
_CQ = _CV // 4
_GRID = -(-_VOCAB // _CV)
_VPAD = _GRID * _CV

_NC = 2
_NS = 16
_NW = _NC * _NS
_BPW = _B // _NW
_C0 = 128
_C1 = _L - _C0
_NBUF = 4
_NIDX = _BPW * _L


def _tr_body(in_ref, out_ref, stg_ref):
    for u in range(_CV // 512):
        s = stg_ref.at[u % 2]
        for q in range(4):
            c0 = 512 * u + 128 * q
            s[32 * q:32 * q + 32, :] = in_ref[:, c0:c0 + 128]
        out_ref[128 * u:128 * u + 128, :] = s[...].T


def _transpose_table(embT):
    return pl.pallas_call(
        _tr_body,
        grid=(_GRID,),
        in_specs=[pl.BlockSpec((_DIM, _CV), lambda k: (0, k))],
        out_specs=pl.BlockSpec((_CQ, 128), lambda k: (k, 0)),
        out_shape=jax.ShapeDtypeStruct((_GRID * _CQ, 128), jnp.float32),
        scratch_shapes=[pltpu.VMEM((2, 128, 128), jnp.float32)],
    )(embT)


def _sc_pool_body(x_hbm, tbl_hbm, out_hbm, idx_v, rows_v, acc_v, sems):
    wid = lax.axis_index("s") * _NC + lax.axis_index("c")
    base = wid * _BPW
    pltpu.sync_copy(x_hbm.at[pl.ds(base * _L, _NIDX)], idx_v)

    def tr_idx(k, carry):
        v = idx_v[pl.ds(k * 16, 16)]
        j = ((v >> 9) << 9) + ((v & 127) << 2) + ((v >> 7) & 3)
        idx_v[pl.ds(k * 16, 16)] = j
        return carry

    lax.fori_loop(0, _NIDX // 16, tr_idx, 0, unroll=8)

    def fire(row, b):
        i0 = row * _L
        pltpu.async_copy(
            tbl_hbm.at[idx_v.at[pl.ds(i0, _C0)]],
            rows_v.at[b].at[pl.ds(0, _C0)], sems.at[b])
        pltpu.async_copy(
            tbl_hbm.at[idx_v.at[pl.ds(i0 + _C0, _C1)]],
            rows_v.at[b].at[pl.ds(_C0, _C1)], sems.at[b])

    def wait(b):
        pltpu.make_async_copy(
            tbl_hbm.at[pl.ds(0, _L)], rows_v.at[b], sems.at[b]).wait()

    def accum(r, b):
        def acc_body(j, accs):
            a0, a1 = accs
            return (a0 + rows_v[b, j, pl.ds(0, 16)],
                    a1 + rows_v[b, j, pl.ds(16, 16)])

        z = jnp.zeros((16,), jnp.float32)
        a0, a1 = lax.fori_loop(0, _L, acc_body, (z, z), unroll=8)
        acc_v[r, pl.ds(0, 16)] = a0
        acc_v[r, pl.ds(16, 16)] = a1

    for b in range(_NBUF):
        fire(b, b)

    def group_body(g, carry):
        for b in range(_NBUF):
            r = g * _NBUF + b
            wait(b)
            accum(r, b)

            @pl.when(r + _NBUF < _BPW)
            def _():
                fire(r + _NBUF, b)
        return carry

    lax.fori_loop(0, _BPW // _NBUF, group_body, 0)
    pltpu.sync_copy(acc_v, out_hbm.at[pl.ds(base, _BPW)])


_sc_pool = functools.partial(
    pl.kernel,
    mesh=plsc.VectorSubcoreMesh(core_axis_name="c", subcore_axis_name="s"),
    out_type=jax.ShapeDtypeStruct((_B, _DIM), jnp.float32),
    compiler_params=pltpu.CompilerParams(use_tc_tiling_on_sc=False),
    scratch_types=[
        pltpu.VMEM((_NIDX,), jnp.int32),
        pltpu.VMEM((_NBUF, _L, _DIM), jnp.float32),
        pltpu.VMEM((_BPW, _DIM), jnp.float32),
        pltpu.SemaphoreType.DMA((_NBUF,)),
    ],
)(_sc_pool_body)


def _mlp_body(m_ref, w1_ref, b1_ref, g_ref, bt_ref, w2_ref, b2_ref, o_ref):
    m = m_ref[...] * (1.0 / _L)
    h = jax.lax.dot_general(
        m, w1_ref[...], (((1,), (0,)), ((), ())),
        preferred_element_type=jnp.float32)
    h = h + b1_ref[...]
    mu = jnp.mean(h, axis=0, keepdims=True)
    d = h - mu
    var = jnp.mean(d * d, axis=0, keepdims=True)
    hn = d * lax.rsqrt(var + _EPS) * g_ref[...] + bt_ref[...]
    hr = jnp.maximum(hn, 0.0)
    o_ref[...] = jax.lax.dot_general(
        hr, w2_ref[...], (((1,), (0,)), ((), ())),
        preferred_element_type=jnp.float32) + b2_ref[...]


def kernel(x, emb, W1, b1, gamma, beta, W2, b2):
    xf = jnp.reshape(x.astype(jnp.int32), (_B * _L,))
    table = _transpose_table(jnp.transpose(emb))
    tblv = jnp.reshape(table, (_VPAD, _DIM))
    msum = _sc_pool(xf, tblv)
    logit = pl.pallas_call(
        _mlp_body,
        out_shape=jax.ShapeDtypeStruct((_B, _CLA), jnp.float32),
    )(msum, W1, b1.reshape(1, _HIDDEN), gamma.reshape(1, _HIDDEN),
      beta.reshape(1, _HIDDEN), W2, b2.reshape(1, _CLA))
    return logit


# speedup vs baseline: 3.6940x; 1.0481x over previous
import functools

import jax
import jax.numpy as jnp
from jax import lax
from jax.experimental import pallas as pl
from jax.experimental.pallas import tpu as pltpu
from jax.experimental.pallas import tpu_sc as plsc

_VOCAB = 1000000
_DIM = 32
_HIDDEN = 128
_CLA = 10
_B = 4096
_L = 200
_EPS = 1e-5

_CV = 65536
_CQ = _CV // 4
_GRID = -(-_VOCAB // _CV)
_VPAD = _GRID * _CV

_NC = 2
_NS = 16
_NW = _NC * _NS
_BPW = _B // _NW
_C0 = 128
_C1 = _L - _C0
_NBUF = 8
_NIDX = _BPW * _L


def _tr_body(in_ref, out_ref, stg_ref):
    for u in range(_CV // 512):
        s = stg_ref.at[u % 2]
        for q in range(4):
            c0 = 512 * u + 128 * q
            s[32 * q:32 * q + 32, :] = in_ref[:, c0:c0 + 128]
        out_ref[128 * u:128 * u + 128, :] = s[...].T


def _transpose_table(embT):
    return pl.pallas_call(
        _tr_body,
        grid=(_GRID,),
        in_specs=[pl.BlockSpec((_DIM, _CV), lambda k: (0, k))],
        out_specs=pl.BlockSpec((_CQ, 128), lambda k: (k, 0)),
        out_shape=jax.ShapeDtypeStruct((_GRID * _CQ, 128), jnp.float32),
        scratch_shapes=[pltpu.VMEM((2, 128, 128), jnp.float32)],
    )(embT)


def _sc_pool_body(x_hbm, tbl_hbm, out_hbm, idx_v, rows_v, acc_v, sems):
    wid = lax.axis_index("s") * _NC + lax.axis_index("c")
    base = wid * _BPW
    pltpu.sync_copy(x_hbm.at[pl.ds(base * _L, _NIDX)], idx_v)

    def tr_idx(k, carry):
        v = idx_v[pl.ds(k * 16, 16)]
        j = ((v >> 9) << 9) + ((v & 127) << 2) + ((v >> 7) & 3)
        idx_v[pl.ds(k * 16, 16)] = j
        return carry

    lax.fori_loop(0, _NIDX // 16, tr_idx, 0, unroll=8)

    def fire(row, b):
        i0 = row * _L
        pltpu.async_copy(
            tbl_hbm.at[idx_v.at[pl.ds(i0, _C0)]],
            rows_v.at[b].at[pl.ds(0, _C0)], sems.at[b])
        pltpu.async_copy(
            tbl_hbm.at[idx_v.at[pl.ds(i0 + _C0, _C1)]],
            rows_v.at[b].at[pl.ds(_C0, _C1)], sems.at[b])

    def wait(b):
        pltpu.make_async_copy(
            tbl_hbm.at[pl.ds(0, _L)], rows_v.at[b], sems.at[b]).wait()

    def accum(r, b):
        def acc_body(j, accs):
            a0, a1 = accs
            return (a0 + rows_v[b, j, pl.ds(0, 16)],
                    a1 + rows_v[b, j, pl.ds(16, 16)])

        z = jnp.zeros((16,), jnp.float32)
        a0, a1 = lax.fori_loop(0, _L, acc_body, (z, z), unroll=8)
        acc_v[r, pl.ds(0, 16)] = a0
        acc_v[r, pl.ds(16, 16)] = a1

    for b in range(_NBUF):
        fire(b, b)

    def group_body(g, carry):
        for b in range(_NBUF):
            r = g * _NBUF + b
            wait(b)
            accum(r, b)

            @pl.when(r + _NBUF < _BPW)
            def _():
                fire(r + _NBUF, b)
        return carry

    lax.fori_loop(0, _BPW // _NBUF, group_body, 0)
    pltpu.sync_copy(acc_v, out_hbm.at[pl.ds(base, _BPW)])


_sc_pool = functools.partial(
    pl.kernel,
    mesh=plsc.VectorSubcoreMesh(core_axis_name="c", subcore_axis_name="s"),
    out_type=jax.ShapeDtypeStruct((_B, _DIM), jnp.float32),
    compiler_params=pltpu.CompilerParams(use_tc_tiling_on_sc=False),
    scratch_types=[
        pltpu.VMEM((_NIDX,), jnp.int32),
        pltpu.VMEM((_NBUF, _L, _DIM), jnp.float32),
        pltpu.VMEM((_BPW, _DIM), jnp.float32),
        pltpu.SemaphoreType.DMA((_NBUF,)),
    ],
)(_sc_pool_body)


def _mlp_body(m_ref, w1_ref, b1_ref, g_ref, bt_ref, w2_ref, b2_ref, o_ref):
    m = m_ref[...] * (1.0 / _L)
    h = jax.lax.dot_general(
        m, w1_ref[...], (((1,), (0,)), ((), ())),
        preferred_element_type=jnp.float32)
    h = h + b1_ref[...]
    mu = jnp.mean(h, axis=0, keepdims=True)
    d = h - mu
    var = jnp.mean(d * d, axis=0, keepdims=True)
    hn = d * lax.rsqrt(var + _EPS) * g_ref[...] + bt_ref[...]
    hr = jnp.maximum(hn, 0.0)
    o_ref[...] = jax.lax.dot_general(
        hr, w2_ref[...], (((1,), (0,)), ((), ())),
        preferred_element_type=jnp.float32) + b2_ref[...]


def kernel(x, emb, W1, b1, gamma, beta, W2, b2):
    xf = jnp.reshape(x.astype(jnp.int32), (_B * _L,))
    table = _transpose_table(jnp.transpose(emb))
    tblv = jnp.reshape(table, (_VPAD, _DIM))
    msum = _sc_pool(xf, tblv)
    logit = pl.pallas_call(
        _mlp_body,
        out_shape=jax.ShapeDtypeStruct((_B, _CLA), jnp.float32),
    )(msum, W1, b1.reshape(1, _HIDDEN), gamma.reshape(1, _HIDDEN),
      beta.reshape(1, _HIDDEN), W2, b2.reshape(1, _CLA))
    return logit
